# Initial kernel scaffold; baseline (speedup 1.0000x reference)
import jax
import jax.numpy as jnp
from jax.experimental import pallas as pl


def kernel(input_ids, puzzle_identifiers, emb_table, sparse_table):
    raise NotImplementedError("write your pallas kernel here")


# speedup vs baseline: 3.9795x; 3.9795x over previous
import functools
import math

import jax
import jax.numpy as jnp
from jax import lax
from jax.experimental import pallas as pl
from jax.experimental.pallas import tpu as pltpu
from jax.experimental.pallas import tpu_sc as plsc

VOCAB = 100000
D = 128
B = 4096
S = 200
SEQ = S + 1
SCALE = 1.0 / math.sqrt(D)

_info = plsc.get_sparse_core_info()
NC, NS, L = _info.num_cores, _info.num_subcores, _info.num_lanes
NW = NC * NS
BPW = B // NW
H = S // 2

_ROWS_BLOCK = 1000


def _scale_table_body(x_ref, o_ref):
    o_ref[...] = x_ref[...] * SCALE


def _make_sc_gather():
    mesh = plsc.VectorSubcoreMesh(core_axis_name="c", subcore_axis_name="s")

    @functools.partial(
        pl.kernel,
        mesh=mesh,
        out_type=jax.ShapeDtypeStruct((B, SEQ, D), jnp.float32),
        scratch_types=[
            pltpu.VMEM((2, H), jnp.int32),
            pltpu.VMEM((SEQ, D), jnp.float32),
            pltpu.VMEM((BPW,), jnp.int32),
            pltpu.VMEM((BPW, D), jnp.float32),
            pltpu.SemaphoreType.DMA,
        ],
    )
    def sc_gather(ids_hbm, pids_hbm, table_hbm, sparse_hbm, out_hbm,
                  idx_v, rows_v, pid_v, prow_v, sem):
        wid = lax.axis_index("s") * NC + lax.axis_index("c")
        base = wid * BPW

        pltpu.sync_copy(pids_hbm.at[pl.ds(base, BPW)], pid_v)
        pltpu.async_copy(sparse_hbm.at[pid_v], prow_v, sem).wait()

        def body(i, carry):
            b = base + i
            pltpu.sync_copy(ids_hbm.at[b], idx_v)
            c1 = pltpu.async_copy(table_hbm.at[idx_v.at[0]],
                                  rows_v.at[pl.ds(1, H)], sem)
            c2 = pltpu.async_copy(table_hbm.at[idx_v.at[1]],
                                  rows_v.at[pl.ds(1 + H, H)], sem)
            c1.wait()
            c2.wait()
            for j in range(D // L):
                rows_v[0, pl.ds(j * L, L)] = prow_v[i, pl.ds(j * L, L)] * SCALE
            pltpu.sync_copy(rows_v, out_hbm.at[b])
            return carry

        lax.fori_loop(0, BPW, body, 0)

    return sc_gather


_sc_gather = _make_sc_gather()


def kernel(input_ids, puzzle_identifiers, emb_table, sparse_table):
    scaled = pl.pallas_call(
        _scale_table_body,
        grid=(VOCAB // _ROWS_BLOCK,),
        in_specs=[pl.BlockSpec((_ROWS_BLOCK, D), lambda i: (i, 0))],
        out_specs=pl.BlockSpec((_ROWS_BLOCK, D), lambda i: (i, 0)),
        out_shape=jax.ShapeDtypeStruct((VOCAB, D), jnp.float32),
    )(emb_table)
    ids3 = input_ids.reshape(B, 2, H)
    return _sc_gather(ids3, puzzle_identifiers, scaled, sparse_table)


# speedup vs baseline: 4.9778x; 1.2509x over previous
import functools
import math

import jax
import jax.numpy as jnp
from jax import lax
from jax.experimental import pallas as pl
from jax.experimental.pallas import tpu as pltpu
from jax.experimental.pallas import tpu_sc as plsc

VOCAB = 100000
D = 128
B = 4096
S = 200
SEQ = S + 1
SCALE = 1.0 / math.sqrt(D)

_info = plsc.get_sparse_core_info()
NC, NS, L = _info.num_cores, _info.num_subcores, _info.num_lanes
NW = NC * NS
BPW = B // NW
H = S // 2
NSLOT = 4

_ROWS_BLOCK = 1000


def _scale_table_body(x_ref, o_ref):
    o_ref[...] = x_ref[...] * SCALE


def _make_sc_gather():
    mesh = plsc.VectorSubcoreMesh(core_axis_name="c", subcore_axis_name="s")

    @functools.partial(
        pl.kernel,
        mesh=mesh,
        out_type=jax.ShapeDtypeStruct((B, SEQ, D), jnp.float32),
        scratch_types=[
            pltpu.VMEM((NSLOT, 2, H), jnp.int32),
            pltpu.VMEM((NSLOT, SEQ, D), jnp.float32),
            pltpu.VMEM((BPW,), jnp.int32),
            pltpu.VMEM((BPW, D), jnp.float32),
            pltpu.SemaphoreType.DMA,
            pltpu.SemaphoreType.DMA,
            pltpu.SemaphoreType.DMA,
            pltpu.SemaphoreType.DMA,
            pltpu.SemaphoreType.DMA,
            pltpu.SemaphoreType.DMA,
            pltpu.SemaphoreType.DMA,
            pltpu.SemaphoreType.DMA,
            pltpu.SemaphoreType.DMA,
            pltpu.SemaphoreType.DMA,
            pltpu.SemaphoreType.DMA,
            pltpu.SemaphoreType.DMA,
            pltpu.SemaphoreType.DMA,
        ],
    )
    def sc_gather(ids_hbm, pids_hbm, table_hbm, sparse_hbm, out_hbm,
                  idx_v, rows_v, pid_v, prow_v, psem,
                  isem0, isem1, isem2, isem3,
                  gsem0, gsem1, gsem2, gsem3,
                  osem0, osem1, osem2, osem3):
        isem = (isem0, isem1, isem2, isem3)
        gsem = (gsem0, gsem1, gsem2, gsem3)
        osem = (osem0, osem1, osem2, osem3)
        wid = lax.axis_index("s") * NC + lax.axis_index("c")
        base = wid * BPW

        pltpu.sync_copy(pids_hbm.at[pl.ds(base, BPW)], pid_v)
        pltpu.async_copy(sparse_hbm.at[pid_v], prow_v, psem).wait()

        def scale_row(i, carry):
            for j in range(D // L):
                prow_v[i, pl.ds(j * L, L)] = prow_v[i, pl.ds(j * L, L)] * SCALE
            return carry

        lax.fori_loop(0, BPW, scale_row, 0)

        def issue_idx(bi, slot):
            return pltpu.async_copy(ids_hbm.at[base + bi], idx_v.at[slot],
                                    isem[slot])

        def wait_idx(slot):
            pltpu.make_async_copy(ids_hbm.at[0], idx_v.at[slot],
                                  isem[slot]).wait()

        def issue_gathers(slot):
            pltpu.async_copy(table_hbm.at[idx_v.at[slot, 0]],
                             rows_v.at[slot, pl.ds(1, H)], gsem[slot])
            pltpu.async_copy(table_hbm.at[idx_v.at[slot, 1]],
                             rows_v.at[slot, pl.ds(1 + H, H)], gsem[slot])

        def wait_gathers(slot):
            pltpu.make_async_copy(table_hbm.at[pl.ds(0, S)],
                                  rows_v.at[slot, pl.ds(1, S)],
                                  gsem[slot]).wait()

        def issue_out(bi, slot):
            pltpu.async_copy(rows_v.at[slot], out_hbm.at[base + bi],
                             osem[slot])

        def wait_out(slot):
            pltpu.make_async_copy(rows_v.at[slot], out_hbm.at[0],
                                  osem[slot]).wait()

        def fill_row0(bi, slot):
            for j in range(D // L):
                rows_v[slot, 0, pl.ds(j * L, L)] = prow_v[bi, pl.ds(j * L, L)]

        issue_idx(0, 0)
        issue_idx(1, 1)
        issue_idx(2, 2)
        wait_idx(0)
        issue_gathers(0)
        wait_idx(1)
        issue_gathers(1)

        def body(i4, carry):
            for p in range(NSLOT):
                i = i4 * NSLOT + p
                q = (p + 2) % NSLOT
                r = (p + 3) % NSLOT
                wait_gathers(p)
                fill_row0(i, p)
                issue_out(i, p)

                @pl.when(i + 3 < BPW)
                def _():
                    issue_idx(i + 3, r)

                @pl.when(i + 2 < BPW)
                def _():
                    @pl.when(i >= 2)
                    def _():
                        wait_out(q)
                    wait_idx(q)
                    issue_gathers(q)
            return carry

        lax.fori_loop(0, BPW // NSLOT, body, 0)
        for t in range(NSLOT):
            wait_out((BPW - NSLOT + t) % NSLOT)

    return sc_gather


_sc_gather = _make_sc_gather()


def kernel(input_ids, puzzle_identifiers, emb_table, sparse_table):
    scaled = pl.pallas_call(
        _scale_table_body,
        grid=(VOCAB // _ROWS_BLOCK,),
        in_specs=[pl.BlockSpec((_ROWS_BLOCK, D), lambda i: (i, 0))],
        out_specs=pl.BlockSpec((_ROWS_BLOCK, D), lambda i: (i, 0)),
        out_shape=jax.ShapeDtypeStruct((VOCAB, D), jnp.float32),
    )(emb_table)
    ids3 = input_ids.reshape(B, 2, H)
    return _sc_gather(ids3, puzzle_identifiers, scaled, sparse_table)


# speedup vs baseline: 5.6304x; 1.1311x over previous
import functools
import math

import jax
import jax.numpy as jnp
from jax import lax
from jax.experimental import pallas as pl
from jax.experimental.pallas import tpu as pltpu
from jax.experimental.pallas import tpu_sc as plsc

VOCAB = 100000
D = 128
B = 4096
S = 200
SEQ = S + 1
SCALE = 1.0 / math.sqrt(D)

_info = plsc.get_sparse_core_info()
NC, NS, L = _info.num_cores, _info.num_subcores, _info.num_lanes
NW = NC * NS
BPW = B // NW
H = S // 2
NSLOT = 4

_ROWS_BLOCK = 1000


def _scale_table_body(x_ref, o_ref):
    o_ref[...] = x_ref[...] * SCALE


def _make_sc_gather():
    mesh = plsc.VectorSubcoreMesh(core_axis_name="c", subcore_axis_name="s")

    @functools.partial(
        pl.kernel,
        mesh=mesh,
        out_type=jax.ShapeDtypeStruct((B, SEQ, D), jnp.float32),
        scratch_types=[
            pltpu.VMEM((NSLOT, 2, H), jnp.int32),
            pltpu.VMEM((NSLOT, SEQ, D), jnp.float32),
            pltpu.VMEM((BPW,), jnp.int32),
            pltpu.VMEM((BPW, D), jnp.float32),
            pltpu.SemaphoreType.DMA,
            pltpu.SemaphoreType.DMA,
            pltpu.SemaphoreType.DMA,
            pltpu.SemaphoreType.DMA,
            pltpu.SemaphoreType.DMA,
            pltpu.SemaphoreType.DMA,
            pltpu.SemaphoreType.DMA,
            pltpu.SemaphoreType.DMA,
            pltpu.SemaphoreType.DMA,
            pltpu.SemaphoreType.DMA,
            pltpu.SemaphoreType.DMA,
            pltpu.SemaphoreType.DMA,
            pltpu.SemaphoreType.DMA,
        ],
    )
    def sc_gather(ids_hbm, pids_hbm, table_hbm, sparse_hbm, out_hbm,
                  idx_v, rows_v, pid_v, prow_v, psem,
                  isem0, isem1, isem2, isem3,
                  gsem0, gsem1, gsem2, gsem3,
                  osem0, osem1, osem2, osem3):
        isem = (isem0, isem1, isem2, isem3)
        gsem = (gsem0, gsem1, gsem2, gsem3)
        osem = (osem0, osem1, osem2, osem3)
        wid = lax.axis_index("s") * NC + lax.axis_index("c")
        base = wid * BPW

        pltpu.sync_copy(pids_hbm.at[pl.ds(base, BPW)], pid_v)
        pltpu.async_copy(sparse_hbm.at[pid_v], prow_v, psem).wait()

        def scale_row(i, carry):
            for j in range(D // L):
                prow_v[i, pl.ds(j * L, L)] = prow_v[i, pl.ds(j * L, L)] * SCALE
            return carry

        lax.fori_loop(0, BPW, scale_row, 0)

        def issue_idx(bi, slot):
            return pltpu.async_copy(ids_hbm.at[base + bi], idx_v.at[slot],
                                    isem[slot])

        def wait_idx(slot):
            pltpu.make_async_copy(ids_hbm.at[0], idx_v.at[slot],
                                  isem[slot]).wait()

        def issue_gathers(slot):
            pltpu.async_copy(table_hbm.at[idx_v.at[slot, 0]],
                             rows_v.at[slot, pl.ds(1, H)], gsem[slot])
            pltpu.async_copy(table_hbm.at[idx_v.at[slot, 1]],
                             rows_v.at[slot, pl.ds(1 + H, H)], gsem[slot])

        def wait_gathers(slot):
            pltpu.make_async_copy(table_hbm.at[pl.ds(0, S)],
                                  rows_v.at[slot, pl.ds(1, S)],
                                  gsem[slot]).wait()

        def issue_out(bi, slot):
            pltpu.async_copy(rows_v.at[slot], out_hbm.at[base + bi],
                             osem[slot])

        def wait_out(slot):
            pltpu.make_async_copy(rows_v.at[slot], out_hbm.at[0],
                                  osem[slot]).wait()

        def fill_row0(bi, slot):
            for j in range(D // L):
                rows_v[slot, 0, pl.ds(j * L, L)] = prow_v[bi, pl.ds(j * L, L)]

        def scale_rows(slot):
            def srow(r, carry):
                for j in range(D // L):
                    rows_v[slot, r, pl.ds(j * L, L)] = (
                        rows_v[slot, r, pl.ds(j * L, L)] * SCALE)
                return carry
            lax.fori_loop(1, SEQ, srow, 0)

        issue_idx(0, 0)
        issue_idx(1, 1)
        issue_idx(2, 2)
        wait_idx(0)
        issue_gathers(0)
        wait_idx(1)
        issue_gathers(1)

        def body(i4, carry):
            for p in range(NSLOT):
                i = i4 * NSLOT + p
                q = (p + 2) % NSLOT
                r = (p + 3) % NSLOT
                wait_gathers(p)
                scale_rows(p)
                fill_row0(i, p)
                issue_out(i, p)

                @pl.when(i + 3 < BPW)
                def _():
                    issue_idx(i + 3, r)

                @pl.when(i + 2 < BPW)
                def _():
                    @pl.when(i >= 2)
                    def _():
                        wait_out(q)
                    wait_idx(q)
                    issue_gathers(q)
            return carry

        lax.fori_loop(0, BPW // NSLOT, body, 0)
        for t in range(NSLOT):
            wait_out((BPW - NSLOT + t) % NSLOT)

    return sc_gather


_sc_gather = _make_sc_gather()


def kernel(input_ids, puzzle_identifiers, emb_table, sparse_table):
    ids3 = input_ids.reshape(B, 2, H)
    return _sc_gather(ids3, puzzle_identifiers, emb_table, sparse_table)


# speedup vs baseline: 8.4490x; 1.5006x over previous
import functools
import math

import jax
import jax.numpy as jnp
from jax import lax
from jax.experimental import pallas as pl
from jax.experimental.pallas import tpu as pltpu
from jax.experimental.pallas import tpu_sc as plsc

VOCAB = 100000
D = 128
B = 4096
S = 200
SEQ = S + 1
SCALE = 1.0 / math.sqrt(D)

_info = plsc.get_sparse_core_info()
NC, NS, L = _info.num_cores, _info.num_subcores, _info.num_lanes
NW = NC * NS
BPW = B // NW
H = S // 2
NSLOT = 4
NA = 112
NB = 96
NR = NA + NB


def _make_sc_gather():
    mesh = plsc.VectorSubcoreMesh(core_axis_name="c", subcore_axis_name="s")

    @functools.partial(
        pl.kernel,
        mesh=mesh,
        out_type=jax.ShapeDtypeStruct((SEQ * B, D), jnp.float32),
        scratch_types=[
            pltpu.VMEM((NSLOT, 2, H), jnp.int32),
            pltpu.VMEM((NSLOT, NR, D), jnp.float32),
            pltpu.VMEM((NSLOT, NA), jnp.int32),
            pltpu.VMEM((NSLOT, NB), jnp.int32),
            pltpu.VMEM((BPW,), jnp.int32),
            pltpu.VMEM((BPW, D), jnp.float32),
            pltpu.SemaphoreType.DMA,
            pltpu.SemaphoreType.DMA,
            pltpu.SemaphoreType.DMA,
            pltpu.SemaphoreType.DMA,
            pltpu.SemaphoreType.DMA,
            pltpu.SemaphoreType.DMA,
            pltpu.SemaphoreType.DMA,
            pltpu.SemaphoreType.DMA,
            pltpu.SemaphoreType.DMA,
            pltpu.SemaphoreType.DMA,
            pltpu.SemaphoreType.DMA,
            pltpu.SemaphoreType.DMA,
            pltpu.SemaphoreType.DMA,
        ],
    )
    def sc_gather(ids_hbm, pids_hbm, table_hbm, sparse_hbm, out_hbm,
                  idx_v, rows_v, didxa_v, didxb_v, pid_v, prow_v, psem,
                  isem0, isem1, isem2, isem3,
                  gsem0, gsem1, gsem2, gsem3,
                  osem0, osem1, osem2, osem3):
        isem = (isem0, isem1, isem2, isem3)
        gsem = (gsem0, gsem1, gsem2, gsem3)
        osem = (osem0, osem1, osem2, osem3)
        wid = lax.axis_index("s") * NC + lax.axis_index("c")
        base = wid * BPW

        pltpu.sync_copy(pids_hbm.at[pl.ds(base, BPW)], pid_v)
        pltpu.async_copy(sparse_hbm.at[pid_v], prow_v, psem).wait()

        def scale_prow(i, carry):
            for j in range(D // L):
                prow_v[i, pl.ds(j * L, L)] = prow_v[i, pl.ds(j * L, L)] * SCALE
            return carry

        lax.fori_loop(0, BPW, scale_prow, 0)

        def issue_idx(bi, slot):
            return pltpu.async_copy(ids_hbm.at[base + bi], idx_v.at[slot],
                                    isem[slot])

        def wait_idx(slot):
            pltpu.make_async_copy(ids_hbm.at[0], idx_v.at[slot],
                                  isem[slot]).wait()

        def issue_gathers(slot):
            pltpu.async_copy(table_hbm.at[idx_v.at[slot, 0]],
                             rows_v.at[slot, pl.ds(0, H)], gsem[slot])
            pltpu.async_copy(table_hbm.at[idx_v.at[slot, 1]],
                             rows_v.at[slot, pl.ds(H, H)], gsem[slot])

        def wait_gathers(slot):
            pltpu.make_async_copy(table_hbm.at[pl.ds(0, S)],
                                  rows_v.at[slot, pl.ds(0, S)],
                                  gsem[slot]).wait()

        def scale_rows(slot):
            def srow(r, carry):
                for j in range(D // L):
                    rows_v[slot, r, pl.ds(j * L, L)] = (
                        rows_v[slot, r, pl.ds(j * L, L)] * SCALE)
                return carry
            lax.fori_loop(0, S, srow, 0)

        lane = lax.iota(jnp.int32, L)

        def fill_didx(bi, slot):
            b = base + bi
            for j in range(NA // L):
                didxa_v[slot, pl.ds(j * L, L)] = (
                    lane + (j * L + 1)) * B + b
            for j in range(NB // L):
                k = lane + (NA + j * L)
                vals = (k + 1) * B + b
                if (j + 1) * L > NB - 8:
                    vals = jnp.where(k < S, vals, base)
                didxb_v[slot, pl.ds(j * L, L)] = vals

        def issue_out(bi, slot):
            pltpu.async_copy(rows_v.at[slot, pl.ds(0, NA)],
                             out_hbm.at[didxa_v.at[slot]], osem[slot])
            pltpu.async_copy(rows_v.at[slot, pl.ds(NA, NB)],
                             out_hbm.at[didxb_v.at[slot]], osem[slot])

        def wait_out(slot):
            pltpu.make_async_copy(rows_v.at[slot],
                                  out_hbm.at[pl.ds(0, NR)],
                                  osem[slot]).wait()

        issue_idx(0, 0)
        issue_idx(1, 1)
        issue_idx(2, 2)
        wait_idx(0)
        issue_gathers(0)
        wait_idx(1)
        issue_gathers(1)

        def body(i4, carry):
            for p in range(NSLOT):
                i = i4 * NSLOT + p
                q = (p + 2) % NSLOT
                r = (p + 3) % NSLOT
                wait_gathers(p)
                scale_rows(p)
                fill_didx(i, p)
                issue_out(i, p)

                @pl.when(i + 3 < BPW)
                def _():
                    issue_idx(i + 3, r)

                @pl.when(i + 2 < BPW)
                def _():
                    @pl.when(i >= 2)
                    def _():
                        wait_out(q)
                    wait_idx(q)
                    issue_gathers(q)
            return carry

        lax.fori_loop(0, BPW // NSLOT, body, 0)
        for t in range(NSLOT):
            wait_out((BPW - NSLOT + t) % NSLOT)

        pltpu.sync_copy(prow_v, out_hbm.at[pl.ds(base, BPW)])

    return sc_gather


_sc_gather = _make_sc_gather()


def kernel(input_ids, puzzle_identifiers, emb_table, sparse_table):
    ids3 = input_ids.reshape(B, 2, H)
    flat = _sc_gather(ids3, puzzle_identifiers, emb_table, sparse_table)
    return flat.reshape(SEQ, B, D).swapaxes(0, 1)


# speedup vs baseline: 10.2796x; 1.2167x over previous
import functools
import math

import jax
import jax.numpy as jnp
from jax import lax
from jax.experimental import pallas as pl
from jax.experimental.pallas import tpu as pltpu
from jax.experimental.pallas import tpu_sc as plsc

VOCAB = 100000
D = 128
B = 4096
S = 200
SEQ = S + 1
SCALE = 1.0 / math.sqrt(D)

_info = plsc.get_sparse_core_info()
NC, NS, L = _info.num_cores, _info.num_subcores, _info.num_lanes
NW = NC * NS
BPW = B // NW
CK = 128
NCHUNK = B // CK
UNITS = S * NCHUNK // NW
NSLOT = 4


def _make_sc_gather():
    mesh = plsc.VectorSubcoreMesh(core_axis_name="c", subcore_axis_name="s")

    @functools.partial(
        pl.kernel,
        mesh=mesh,
        out_type=jax.ShapeDtypeStruct((SEQ * B, D), jnp.float32),
        scratch_types=[
            pltpu.VMEM((NSLOT, CK), jnp.int32),
            pltpu.VMEM((NSLOT, CK, D), jnp.float32),
            pltpu.VMEM((BPW,), jnp.int32),
            pltpu.VMEM((BPW, D), jnp.float32),
            pltpu.SemaphoreType.DMA,
            pltpu.SemaphoreType.DMA,
            pltpu.SemaphoreType.DMA,
            pltpu.SemaphoreType.DMA,
            pltpu.SemaphoreType.DMA,
            pltpu.SemaphoreType.DMA,
            pltpu.SemaphoreType.DMA,
            pltpu.SemaphoreType.DMA,
            pltpu.SemaphoreType.DMA,
            pltpu.SemaphoreType.DMA,
            pltpu.SemaphoreType.DMA,
            pltpu.SemaphoreType.DMA,
            pltpu.SemaphoreType.DMA,
        ],
    )
    def sc_gather(ids_t_hbm, pids_hbm, table_hbm, sparse_hbm, out_hbm,
                  idx_v, rows_v, pid_v, prow_v, psem,
                  isem0, isem1, isem2, isem3,
                  gsem0, gsem1, gsem2, gsem3,
                  osem0, osem1, osem2, osem3):
        isem = (isem0, isem1, isem2, isem3)
        gsem = (gsem0, gsem1, gsem2, gsem3)
        osem = (osem0, osem1, osem2, osem3)
        wid = lax.axis_index("s") * NC + lax.axis_index("c")
        base = wid * BPW
        u0 = wid * UNITS

        pltpu.sync_copy(pids_hbm.at[pl.ds(base, BPW)], pid_v)
        pltpu.async_copy(sparse_hbm.at[pid_v], prow_v, psem).wait()

        def scale_prow(i, carry):
            for j in range(D // L):
                prow_v[i, pl.ds(j * L, L)] = prow_v[i, pl.ds(j * L, L)] * SCALE
            return carry

        lax.fori_loop(0, BPW, scale_prow, 0)
        pltpu.sync_copy(prow_v, out_hbm.at[pl.ds(base, BPW)])

        def issue_idx(t, slot):
            u = u0 + t
            s = u // NCHUNK
            c = u % NCHUNK
            pltpu.async_copy(ids_t_hbm.at[s, pl.ds(c * CK, CK)],
                             idx_v.at[slot], isem[slot])

        def wait_idx(slot):
            pltpu.make_async_copy(ids_t_hbm.at[0, pl.ds(0, CK)],
                                  idx_v.at[slot], isem[slot]).wait()

        def issue_gather(slot):
            pltpu.async_copy(table_hbm.at[idx_v.at[slot]],
                             rows_v.at[slot], gsem[slot])

        def wait_gather(slot):
            pltpu.make_async_copy(table_hbm.at[pl.ds(0, CK)],
                                  rows_v.at[slot], gsem[slot]).wait()

        def scale_rows(slot):
            def srow(r, carry):
                for j in range(D // L):
                    rows_v[slot, r, pl.ds(j * L, L)] = (
                        rows_v[slot, r, pl.ds(j * L, L)] * SCALE)
                return carry
            lax.fori_loop(0, CK, srow, 0)

        def issue_out(t, slot):
            off = pl.multiple_of(B + (u0 + t) * CK, CK)
            pltpu.async_copy(rows_v.at[slot], out_hbm.at[pl.ds(off, CK)],
                             osem[slot])

        def wait_out(slot):
            pltpu.make_async_copy(rows_v.at[slot],
                                  out_hbm.at[pl.ds(0, CK)],
                                  osem[slot]).wait()

        issue_idx(0, 0)
        issue_idx(1, 1)
        issue_idx(2, 2)
        wait_idx(0)
        issue_gather(0)
        wait_idx(1)
        issue_gather(1)

        def body(i4, carry):
            for p in range(NSLOT):
                t = i4 * NSLOT + p
                q = (p + 2) % NSLOT
                r = (p + 3) % NSLOT
                wait_gather(p)
                scale_rows(p)
                issue_out(t, p)

                @pl.when(t + 3 < UNITS)
                def _():
                    issue_idx(t + 3, r)

                @pl.when(t + 2 < UNITS)
                def _():
                    @pl.when(t >= 2)
                    def _():
                        wait_out(q)
                    wait_idx(q)
                    issue_gather(q)
            return carry

        lax.fori_loop(0, UNITS // NSLOT, body, 0)
        for t in range(NSLOT):
            wait_out((UNITS - NSLOT + t) % NSLOT)

    return sc_gather


_sc_gather = _make_sc_gather()


def kernel(input_ids, puzzle_identifiers, emb_table, sparse_table):
    ids_t = input_ids.T
    flat = _sc_gather(ids_t, puzzle_identifiers, emb_table, sparse_table)
    return flat.reshape(SEQ, B, D).swapaxes(0, 1)
